# baseline (speedup 1.0000x reference)
import functools

import jax
import jax.numpy as jnp
from jax import lax
from jax.experimental import pallas as pl
from jax.experimental.pallas import tpu as pltpu
from jax.experimental.pallas import tpu_sc as plsc

TOTAL = 32768
D = 512
BS = 16
PER_BATCH = TOTAL // BS
NC, NS = 2, 16
NW = NC * NS
TOK_W = TOTAL // NW
LANES = 16

ACHUNK = 64
NA = (TOK_W // 2) // ACHUNK
BCHUNK = 32
NB = (TOK_W // 2) // BCHUNK


def _make_padded_scatter():
    mesh = plsc.VectorSubcoreMesh(core_axis_name="c", subcore_axis_name="s")

    @functools.partial(
        pl.kernel,
        mesh=mesh,
        out_type=jax.ShapeDtypeStruct((TOTAL, D), jnp.float32),
        scratch_types=[
            pltpu.VMEM((TOK_W,), jnp.int32),
            pltpu.VMEM((NA, ACHUNK), jnp.int32),
            pltpu.VMEM((ACHUNK, D), jnp.float32),
            pltpu.VMEM((ACHUNK, D), jnp.float32),
            pltpu.VMEM_SHARED((NS, 2, BCHUNK, D), jnp.float32),
        ]
        + [pltpu.SemaphoreType.DMA] * 8,
    )
    def padded_scatter(feat_hbm, idx_hbm, out_hbm, idx_blk, dst_all,
                       abuf0, abuf1, spm, *sems):
        wid = lax.axis_index("s") * NC + lax.axis_index("c")
        sid = lax.axis_index("s")
        base = wid * TOK_W
        bbase = base + TOK_W // 2
        iota = lax.iota(jnp.int32, LANES)

        abufs = (abuf0, abuf1)
        agsems, assems = sems[0:2], sems[2:4]
        bgsems, bssems = sems[4:6], sems[6:8]

        pltpu.sync_copy(idx_hbm.at[pl.ds(base, TOK_W)], idx_blk)
        for c in range(NA):
            for j in range(ACHUNK // LANES):
                tok = c * ACHUNK + j * LANES
                gpos = iota + (base + tok)
                bid = idx_blk[pl.ds(tok, LANES)]
                dst = bid * PER_BATCH + (gpos & (PER_BATCH - 1))
                dst_all.at[c][pl.ds(j * LANES, LANES)] = dst

        ag = [None, None]
        asc = [None, None]
        bg = [None, None]
        bsc = [None, None]

        def a_gather(c):
            return pltpu.async_copy(
                feat_hbm.at[pl.ds(base + c * ACHUNK, ACHUNK), :],
                abufs[c % 2], agsems[c % 2])

        def b_gather(c):
            return pltpu.async_copy(
                feat_hbm.at[pl.ds(bbase + c * BCHUNK, BCHUNK), :],
                spm.at[sid, c % 2], bgsems[c % 2])

        def a_iter(i):
            k = i % 2
            nk = (i + 1) % 2
            if i + 1 < NA:
                if i >= 1:
                    asc[nk].wait()
                ag[nk] = a_gather(i + 1)
            ag[k].wait()
            asc[k] = pltpu.async_copy(abufs[k], out_hbm.at[dst_all.at[i]],
                                      assems[k])

        def b_iter(i):
            k = i % 2
            nk = (i + 1) % 2
            if i + 1 < NB:
                if i >= 1:
                    bsc[nk].wait()
                bg[nk] = b_gather(i + 1)
            bg[k].wait()
            bsc[k] = pltpu.async_copy(
                spm.at[sid, k],
                out_hbm.at[pl.ds(bbase + i * BCHUNK, BCHUNK), :],
                bssems[k])

        ag[0] = a_gather(0)
        bg[0] = b_gather(0)
        for i in range(NB):
            if i % 2 == 0:
                a_iter(i // 2)
            b_iter(i)
        asc[0].wait()
        asc[1].wait()
        bsc[0].wait()
        bsc[1].wait()

    return padded_scatter


_PADDED_SCATTER = _make_padded_scatter()


def kernel(features, indices, batch_size):
    del batch_size
    col0 = indices[:, 0].astype(jnp.int32)
    out = _PADDED_SCATTER(features, col0)
    return out.reshape(BS, PER_BATCH, D)
